# baseline (speedup 1.0000x reference)
import functools

import jax
import jax.numpy as jnp
from jax import lax
from jax.experimental import pallas as pl
from jax.experimental.pallas import tpu as pltpu
from jax.experimental.pallas import tpu_sc as plsc

B = 16384
L = 200
NUM_EMB = 100
EMB_DIM = 64

_NW = 32
_LT = L // 8
_BT = B // 128
_BC_PER_W = _BT // _NW
_TR_CHUNK = 5
_N_CHUNKS = _LT // _TR_CHUNK
_PAD_EMB = 128


def _sc_lookup(ci4, emb_t, wb):
    mesh = plsc.VectorSubcoreMesh(core_axis_name="c", subcore_axis_name="s")

    @functools.partial(
        pl.kernel,
        mesh=mesh,
        out_type=jax.ShapeDtypeStruct((L, 1, B), jnp.float32),
        compiler_params=pltpu.CompilerParams(
            needs_layout_passes=False,
            use_tc_tiling_on_sc=False,
            disable_bounds_checks=True,
        ),
        scratch_types=[
            pltpu.VMEM((EMB_DIM, _PAD_EMB), jnp.float32),
            pltpu.VMEM((80,), jnp.float32),
            pltpu.VMEM((_PAD_EMB,), jnp.float32),
            pltpu.VMEM((_TR_CHUNK, _BC_PER_W, 8, 128), jnp.int32),
            pltpu.VMEM((_TR_CHUNK, _BC_PER_W, 8, 128), jnp.int32),
            pltpu.VMEM((_TR_CHUNK * 8, 1, _BC_PER_W * 128), jnp.float32),
            pltpu.VMEM((_TR_CHUNK * 8, 1, _BC_PER_W * 128), jnp.float32),
            pltpu.SemaphoreType.DMA,
            pltpu.SemaphoreType.DMA,
            pltpu.SemaphoreType.DMA,
            pltpu.SemaphoreType.DMA,
        ],
    )
    def k(ci_hbm, embt_hbm, wb_hbm, out_hbm, embt_v, wb_v, scores_v,
          idx0_v, idx1_v, out0_v, out1_v, isem0, isem1, osem0, osem1):
        wid = lax.axis_index("s") * 2 + lax.axis_index("c")
        bc0 = wid * _BC_PER_W

        def in_slice(c):
            return ci_hbm.at[pl.ds(c * _TR_CHUNK, _TR_CHUNK),
                             pl.ds(bc0, _BC_PER_W)]

        idx_bufs = (idx0_v, idx1_v)
        isems = (isem0, isem1)
        in_dma = {0: pltpu.async_copy(in_slice(0), idx_bufs[0], isems[0])}

        pltpu.sync_copy(embt_hbm, embt_v)
        pltpu.sync_copy(wb_hbm, wb_v)
        bias = wb_v[pl.ds(EMB_DIM, 16)][0]
        wvecs = [wb_v[pl.ds(g * 16, 16)] for g in range(EMB_DIM // 16)]
        for j in range(_PAD_EMB // 16):
            acc = jnp.full((16,), bias, jnp.float32)
            for d in range(EMB_DIM):
                acc = acc + embt_v[d, pl.ds(j * 16, 16)] * wvecs[d // 16][d % 16]
            scores_v[pl.ds(j * 16, 16)] = acc

        out_bufs = (out0_v, out1_v)
        osems = (osem0, osem1)

        def out_slice(c):
            return out_hbm.at[pl.ds(c * _TR_CHUNK * 8, _TR_CHUNK * 8),
                              pl.ds(0, 1),
                              pl.ds(bc0 * 128, _BC_PER_W * 128)]

        out_dma = {}
        for c in range(_N_CHUNKS):
            b = c % 2
            if c + 1 < _N_CHUNKS:
                in_dma[c + 1] = pltpu.async_copy(
                    in_slice(c + 1), idx_bufs[1 - b], isems[1 - b]
                )
            in_dma[c].wait()
            if c >= 2:
                out_dma[c - 2].wait()
            idx_v, out_v = idx_bufs[b], out_bufs[b]

            @plsc.parallel_loop(0, _TR_CHUNK * 8, 1, unroll=2)
            def gather_body(row):
                tr = row >> 3
                sub = row & 7
                for bc in range(_BC_PER_W):
                    for j in range(8):
                        col = bc * 128 + j * 16
                        out_v[row, 0, pl.ds(col, 16)] = plsc.load_gather(
                            scores_v,
                            [idx_v[tr, bc, sub, pl.ds(j * 16, 16)]],
                        )

            out_dma[c] = pltpu.async_copy(out_v, out_slice(c), osems[b])
        for c in range(max(0, _N_CHUNKS - 2), _N_CHUNKS):
            out_dma[c].wait()

    return k(ci4, emb_t, wb)


def kernel(char_input, embedding, linear_w, linear_b):
    ci = char_input.astype(jnp.int32)
    ci4 = ci.reshape(B // 128, 128, L // 8, 8).transpose(2, 0, 3, 1)
    emb_t = jnp.zeros((EMB_DIM, _PAD_EMB), jnp.float32)
    emb_t = emb_t.at[:, :NUM_EMB].set(embedding.astype(jnp.float32).T)
    wb = jnp.zeros((80,), jnp.float32)
    wb = wb.at[:EMB_DIM].set(linear_w.astype(jnp.float32).reshape(EMB_DIM))
    wb = wb.at[EMB_DIM].set(linear_b.astype(jnp.float32)[0])
    out3 = _sc_lookup(ci4, emb_t, wb)
    return out3.transpose(2, 0, 1)


# speedup vs baseline: 1.0403x; 1.0403x over previous
import functools

import jax
import jax.numpy as jnp
from jax import lax
from jax.experimental import pallas as pl
from jax.experimental.pallas import tpu as pltpu
from jax.experimental.pallas import tpu_sc as plsc

B = 16384
L = 200
NUM_EMB = 100
EMB_DIM = 64

_NW = 32
_LT = L // 8
_BT = B // 128
_BC_PER_W = _BT // _NW
_TR_CHUNK = 5
_N_CHUNKS = _LT // _TR_CHUNK
_PAD_EMB = 128


def _sc_lookup(ci4, emb_t, wb):
    mesh = plsc.VectorSubcoreMesh(core_axis_name="c", subcore_axis_name="s")

    @functools.partial(
        pl.kernel,
        mesh=mesh,
        out_type=jax.ShapeDtypeStruct((L, 1, B), jnp.float32),
        compiler_params=pltpu.CompilerParams(
            needs_layout_passes=False,
            use_tc_tiling_on_sc=False,
            disable_bounds_checks=True,
        ),
        scratch_types=[
            pltpu.VMEM((EMB_DIM, _PAD_EMB), jnp.float32),
            pltpu.VMEM((80,), jnp.float32),
            pltpu.VMEM((_PAD_EMB,), jnp.float32),
            pltpu.VMEM((_TR_CHUNK, _BC_PER_W, 8, 128), jnp.int32),
            pltpu.VMEM((_TR_CHUNK, _BC_PER_W, 8, 128), jnp.int32),
            pltpu.VMEM((_TR_CHUNK * 8, 1, _BC_PER_W * 128), jnp.float32),
            pltpu.VMEM((_TR_CHUNK * 8, 1, _BC_PER_W * 128), jnp.float32),
            pltpu.SemaphoreType.DMA,
            pltpu.SemaphoreType.DMA,
            pltpu.SemaphoreType.DMA,
            pltpu.SemaphoreType.DMA,
        ],
    )
    def k(ci_hbm, embt_hbm, wb_hbm, out_hbm, embt_v, wb_v, scores_v,
          idx0_v, idx1_v, out0_v, out1_v, isem0, isem1, osem0, osem1):
        wid = lax.axis_index("s") * 2 + lax.axis_index("c")
        bc0 = wid * _BC_PER_W

        def in_slice(c):
            return ci_hbm.at[pl.ds(c * _TR_CHUNK, _TR_CHUNK),
                             pl.ds(bc0, _BC_PER_W)]

        idx_bufs = (idx0_v, idx1_v)
        isems = (isem0, isem1)
        in_dma = {0: pltpu.async_copy(in_slice(0), idx_bufs[0], isems[0])}

        pltpu.sync_copy(embt_hbm, embt_v)
        pltpu.sync_copy(wb_hbm, wb_v)
        bias = wb_v[pl.ds(EMB_DIM, 16)][0]
        wvecs = [wb_v[pl.ds(g * 16, 16)] for g in range(EMB_DIM // 16)]
        for j in range(_PAD_EMB // 16):
            acc = jnp.full((16,), bias, jnp.float32)
            for d in range(EMB_DIM):
                acc = acc + embt_v[d, pl.ds(j * 16, 16)] * wvecs[d // 16][d % 16]
            scores_v[pl.ds(j * 16, 16)] = acc

        out_bufs = (out0_v, out1_v)
        osems = (osem0, osem1)

        def out_slice(c):
            return out_hbm.at[pl.ds(c * _TR_CHUNK * 8, _TR_CHUNK * 8),
                              pl.ds(0, 1),
                              pl.ds(bc0 * 128, _BC_PER_W * 128)]

        out_dma = {}
        for c in range(_N_CHUNKS):
            b = c % 2
            if c + 1 < _N_CHUNKS:
                in_dma[c + 1] = pltpu.async_copy(
                    in_slice(c + 1), idx_bufs[1 - b], isems[1 - b]
                )
            in_dma[c].wait()
            if c >= 2:
                out_dma[c - 2].wait()
            idx_v, out_v = idx_bufs[b], out_bufs[b]

            @plsc.parallel_loop(0, _TR_CHUNK * 8, 1)
            def gather_body(row):
                tr = row >> 3
                sub = row & 7
                for bc in range(_BC_PER_W):
                    for j in range(8):
                        col = bc * 128 + j * 16
                        out_v[row, 0, pl.ds(col, 16)] = plsc.load_gather(
                            scores_v,
                            [idx_v[tr, bc, sub, pl.ds(j * 16, 16)]],
                        )

            out_dma[c] = pltpu.async_copy(out_v, out_slice(c), osems[b])
        for c in range(max(0, _N_CHUNKS - 2), _N_CHUNKS):
            out_dma[c].wait()

    return k(ci4, emb_t, wb)


def kernel(char_input, embedding, linear_w, linear_b):
    ci = char_input.astype(jnp.int32)
    ci4 = ci.reshape(B // 128, 128, L // 8, 8).transpose(2, 0, 3, 1)
    emb_t = jnp.zeros((EMB_DIM, _PAD_EMB), jnp.float32)
    emb_t = emb_t.at[:, :NUM_EMB].set(embedding.astype(jnp.float32).T)
    wb = jnp.zeros((80,), jnp.float32)
    wb = wb.at[:EMB_DIM].set(linear_w.astype(jnp.float32).reshape(EMB_DIM))
    wb = wb.at[EMB_DIM].set(linear_b.astype(jnp.float32)[0])
    out3 = _sc_lookup(ci4, emb_t, wb)
    return out3.transpose(2, 0, 1)
